# baseline (speedup 1.0000x reference)
import functools

import jax
import jax.numpy as jnp
from jax import lax
from jax.experimental import pallas as pl
from jax.experimental.pallas import tpu as pltpu
from jax.experimental.pallas import tpu_sc as plsc

D = 128


@functools.lru_cache(maxsize=None)
def _gmf_kernel(B, b_per_w, ch, nc):
  n_ch = b_per_w // ch
  mesh = plsc.VectorSubcoreMesh(core_axis_name="c", subcore_axis_name="s")

  @functools.partial(
      pl.kernel,
      mesh=mesh,
      compiler_params=pltpu.CompilerParams(needs_layout_passes=False),
      out_type=jax.ShapeDtypeStruct((B,), jnp.float32),
      scratch_types=[
          pltpu.VMEM((b_per_w,), jnp.int32),
          pltpu.VMEM((b_per_w,), jnp.int32),
          pltpu.VMEM((2, ch, D), jnp.float32),
          pltpu.VMEM((2, ch, D), jnp.float32),
          pltpu.VMEM((128,), jnp.float32),
          pltpu.VMEM((16,), jnp.float32),
          pltpu.VMEM((ch * 16,), jnp.float32),
          pltpu.VMEM((b_per_w,), jnp.float32),
          pltpu.SemaphoreType.DMA,
          pltpu.SemaphoreType.DMA,
          pltpu.SemaphoreType.DMA,
          pltpu.SemaphoreType.DMA,
      ],
  )
  def k(users_hbm, items_hbm, ut_hbm, it_hbm, w_hbm, b_hbm, out_hbm,
        uidx, iidx, ubuf, vbuf, wb, bb, accbuf, outv,
        sem_u0, sem_u1, sem_v0, sem_v1):
    wid = lax.axis_index("s") * nc + lax.axis_index("c")
    base = wid * b_per_w
    cu = pltpu.async_copy(users_hbm.at[pl.ds(base, b_per_w)], uidx, sem_u0)
    cv = pltpu.async_copy(items_hbm.at[pl.ds(base, b_per_w)], iidx, sem_v0)
    cu.wait()
    cv.wait()
    sems_u = (sem_u0, sem_u1)
    sems_v = (sem_v0, sem_v1)

    def start(c, p):
      pltpu.async_copy(
          ut_hbm.at[uidx.at[pl.ds(c * ch, ch)]], ubuf.at[p], sems_u[p])
      pltpu.async_copy(
          it_hbm.at[iidx.at[pl.ds(c * ch, ch)]], vbuf.at[p], sems_v[p])

    def wait(p):
      pltpu.make_async_copy(
          ut_hbm.at[uidx.at[pl.ds(0, ch)]], ubuf.at[p], sems_u[p]).wait()
      pltpu.make_async_copy(
          it_hbm.at[iidx.at[pl.ds(0, ch)]], vbuf.at[p], sems_v[p]).wait()

    start(0, 0)
    start(1, 1)
    pltpu.sync_copy(w_hbm, wb)
    pltpu.sync_copy(b_hbm, bb)
    wvecs = [wb[pl.ds(16 * j, 16)] for j in range(8)]
    col16 = lax.iota(jnp.int32, 16) * 16
    bias_vec = bb[pl.ds(0, 16)]

    def compute(c, p):
      urows = ubuf.at[p]
      vrows = vbuf.at[p]

      @plsc.parallel_loop(0, ch, step=1, unroll=4)
      def _(i):
        acc = urows[i, pl.ds(0, 16)] * vrows[i, pl.ds(0, 16)] * wvecs[0]
        for j in range(1, 8):
          acc = acc + urows[i, pl.ds(16 * j, 16)] * vrows[i, pl.ds(16 * j, 16)] * wvecs[j]
        accbuf[pl.ds(i * 16, 16)] = acc

      @plsc.parallel_loop(0, ch // 16, step=1, unroll=2)
      def _(g):
        gbase = g * 256
        s = plsc.load_gather(accbuf, [col16 + gbase])
        for l in range(1, 16):
          s = s + plsc.load_gather(accbuf, [col16 + (gbase + l)])
        outv[pl.ds(c * ch + g * 16, 16)] = s + bias_vec

    n_pairs = n_ch // 2

    def pair_body(q, _):
      c0 = 2 * q
      wait(0)
      compute(c0, 0)

      @pl.when(q < n_pairs - 1)
      def _():
        start(c0 + 2, 0)

      wait(1)
      compute(c0 + 1, 1)

      @pl.when(q < n_pairs - 1)
      def _():
        start(c0 + 3, 1)

      return 0

    lax.fori_loop(0, n_pairs, pair_body, 0)
    pltpu.sync_copy(outv, out_hbm.at[pl.ds(base, b_per_w)])

  return k


def kernel(users, items, user_table, item_table, W, b):
  B = users.shape[0]
  info = plsc.get_sparse_core_info()
  nw = info.num_cores * info.num_subcores
  k = _gmf_kernel(B, B // nw, 128, info.num_cores)
  return k(users.astype(jnp.int32), items.astype(jnp.int32),
           user_table, item_table, W.reshape(-1),
           jnp.broadcast_to(b, (16,)))


# speedup vs baseline: 1.0123x; 1.0123x over previous
import functools

import jax
import jax.numpy as jnp
from jax import lax
from jax.experimental import pallas as pl
from jax.experimental.pallas import tpu as pltpu
from jax.experimental.pallas import tpu_sc as plsc

D = 128


@functools.lru_cache(maxsize=None)
def _gmf_kernel(B, b_per_w, ch, nc):
  n_ch = b_per_w // ch
  mesh = plsc.VectorSubcoreMesh(core_axis_name="c", subcore_axis_name="s")

  @functools.partial(
      pl.kernel,
      mesh=mesh,
      compiler_params=pltpu.CompilerParams(needs_layout_passes=False),
      out_type=jax.ShapeDtypeStruct((B,), jnp.float32),
      scratch_types=[
          pltpu.VMEM((b_per_w,), jnp.int32),
          pltpu.VMEM((b_per_w,), jnp.int32),
          pltpu.VMEM((2, ch, D), jnp.float32),
          pltpu.VMEM((2, ch, D), jnp.float32),
          pltpu.VMEM((128,), jnp.float32),
          pltpu.VMEM((16,), jnp.float32),
          pltpu.VMEM((ch * 16,), jnp.float32),
          pltpu.VMEM((b_per_w,), jnp.float32),
          pltpu.SemaphoreType.DMA,
          pltpu.SemaphoreType.DMA,
          pltpu.SemaphoreType.DMA,
          pltpu.SemaphoreType.DMA,
      ],
  )
  def k(users_hbm, items_hbm, ut_hbm, it_hbm, w_hbm, b_hbm, out_hbm,
        uidx, iidx, ubuf, vbuf, wb, bb, accbuf, outv,
        sem_u0, sem_u1, sem_v0, sem_v1):
    wid = lax.axis_index("s") * nc + lax.axis_index("c")
    base = wid * b_per_w
    cu0 = pltpu.async_copy(users_hbm.at[pl.ds(base, ch)], uidx.at[pl.ds(0, ch)], sem_u0)
    cv0 = pltpu.async_copy(items_hbm.at[pl.ds(base, ch)], iidx.at[pl.ds(0, ch)], sem_v0)
    rest = b_per_w - ch
    cu1 = pltpu.async_copy(
        users_hbm.at[pl.ds(base + ch, rest)], uidx.at[pl.ds(ch, rest)], sem_u1)
    cv1 = pltpu.async_copy(
        items_hbm.at[pl.ds(base + ch, rest)], iidx.at[pl.ds(ch, rest)], sem_v1)
    cu0.wait()
    cv0.wait()
    sems_u = (sem_u0, sem_u1)
    sems_v = (sem_v0, sem_v1)

    def start(c, p):
      pltpu.async_copy(
          ut_hbm.at[uidx.at[pl.ds(c * ch, ch)]], ubuf.at[p], sems_u[p])
      pltpu.async_copy(
          it_hbm.at[iidx.at[pl.ds(c * ch, ch)]], vbuf.at[p], sems_v[p])

    def wait(p):
      pltpu.make_async_copy(
          ut_hbm.at[uidx.at[pl.ds(0, ch)]], ubuf.at[p], sems_u[p]).wait()
      pltpu.make_async_copy(
          it_hbm.at[iidx.at[pl.ds(0, ch)]], vbuf.at[p], sems_v[p]).wait()

    start(0, 0)
    cu1.wait()
    cv1.wait()
    start(1, 1)
    pltpu.sync_copy(w_hbm, wb)
    pltpu.sync_copy(b_hbm, bb)
    wvecs = [wb[pl.ds(16 * j, 16)] for j in range(8)]
    col16 = lax.iota(jnp.int32, 16) * 16
    bias_vec = bb[pl.ds(0, 16)]

    def compute(c, p):
      urows = ubuf.at[p]
      vrows = vbuf.at[p]

      @plsc.parallel_loop(0, ch, step=1, unroll=2)
      def _(i):
        acc = urows[i, pl.ds(0, 16)] * vrows[i, pl.ds(0, 16)] * wvecs[0]
        for j in range(1, 8):
          acc = acc + urows[i, pl.ds(16 * j, 16)] * vrows[i, pl.ds(16 * j, 16)] * wvecs[j]
        accbuf[pl.ds(i * 16, 16)] = acc

      @plsc.parallel_loop(0, ch // 16, step=1)
      def _(g):
        gbase = g * 256
        s = plsc.load_gather(accbuf, [col16 + gbase])
        for l in range(1, 16):
          s = s + plsc.load_gather(accbuf, [col16 + (gbase + l)])
        outv[pl.ds(c * ch + g * 16, 16)] = s + bias_vec

    n_pairs = n_ch // 2

    def pair_body(q, _):
      c0 = 2 * q
      wait(0)
      compute(c0, 0)

      @pl.when(q < n_pairs - 1)
      def _():
        start(c0 + 2, 0)

      wait(1)
      compute(c0 + 1, 1)

      @pl.when(q < n_pairs - 1)
      def _():
        start(c0 + 3, 1)

      return 0

    lax.fori_loop(0, n_pairs, pair_body, 0)
    pltpu.sync_copy(outv, out_hbm.at[pl.ds(base, b_per_w)])

  return k


def kernel(users, items, user_table, item_table, W, b):
  B = users.shape[0]
  info = plsc.get_sparse_core_info()
  nw = info.num_cores * info.num_subcores
  k = _gmf_kernel(B, B // nw, 128, info.num_cores)
  return k(users.astype(jnp.int32), items.astype(jnp.int32),
           user_table, item_table, W.reshape(-1),
           jnp.broadcast_to(b, (16,)))


# speedup vs baseline: 1.0150x; 1.0027x over previous
import functools

import jax
import jax.numpy as jnp
from jax import lax
from jax.experimental import pallas as pl
from jax.experimental.pallas import tpu as pltpu
from jax.experimental.pallas import tpu_sc as plsc

D = 128


@functools.lru_cache(maxsize=None)
def _gmf_kernel(B, b_per_w, ch, nc):
  n_ch = b_per_w // ch
  mesh = plsc.VectorSubcoreMesh(core_axis_name="c", subcore_axis_name="s")

  @functools.partial(
      pl.kernel,
      mesh=mesh,
      compiler_params=pltpu.CompilerParams(needs_layout_passes=False),
      out_type=jax.ShapeDtypeStruct((B,), jnp.float32),
      scratch_types=[
          pltpu.VMEM((b_per_w,), jnp.int32),
          pltpu.VMEM((b_per_w,), jnp.int32),
          pltpu.VMEM((2, ch, D), jnp.float32),
          pltpu.VMEM((2, ch, D), jnp.float32),
          pltpu.VMEM((128,), jnp.float32),
          pltpu.VMEM((16,), jnp.float32),
          pltpu.VMEM((ch * 16,), jnp.float32),
          pltpu.VMEM((b_per_w,), jnp.float32),
          pltpu.SemaphoreType.DMA,
          pltpu.SemaphoreType.DMA,
          pltpu.SemaphoreType.DMA,
          pltpu.SemaphoreType.DMA,
      ],
  )
  def k(users_hbm, items_hbm, ut_hbm, it_hbm, w_hbm, b_hbm, out_hbm,
        uidx, iidx, ubuf, vbuf, wb, bb, accbuf, outv,
        sem_u0, sem_u1, sem_v0, sem_v1):
    wid = lax.axis_index("s") * nc + lax.axis_index("c")
    base = wid * b_per_w
    cu0 = pltpu.async_copy(users_hbm.at[pl.ds(base, ch)], uidx.at[pl.ds(0, ch)], sem_u0)
    cv0 = pltpu.async_copy(items_hbm.at[pl.ds(base, ch)], iidx.at[pl.ds(0, ch)], sem_v0)
    rest = b_per_w - ch
    cu1 = pltpu.async_copy(
        users_hbm.at[pl.ds(base + ch, rest)], uidx.at[pl.ds(ch, rest)], sem_u1)
    cv1 = pltpu.async_copy(
        items_hbm.at[pl.ds(base + ch, rest)], iidx.at[pl.ds(ch, rest)], sem_v1)
    cu0.wait()
    cv0.wait()
    sems_u = (sem_u0, sem_u1)
    sems_v = (sem_v0, sem_v1)

    def start(c, p):
      pltpu.async_copy(
          ut_hbm.at[uidx.at[pl.ds(c * ch, ch)]], ubuf.at[p], sems_u[p])
      pltpu.async_copy(
          it_hbm.at[iidx.at[pl.ds(c * ch, ch)]], vbuf.at[p], sems_v[p])

    def wait(p):
      pltpu.make_async_copy(
          ut_hbm.at[uidx.at[pl.ds(0, ch)]], ubuf.at[p], sems_u[p]).wait()
      pltpu.make_async_copy(
          it_hbm.at[iidx.at[pl.ds(0, ch)]], vbuf.at[p], sems_v[p]).wait()

    start(0, 0)
    cu1.wait()
    cv1.wait()
    start(1, 1)
    pltpu.sync_copy(w_hbm, wb)
    pltpu.sync_copy(b_hbm, bb)
    wvecs = [wb[pl.ds(16 * j, 16)] for j in range(8)]
    col16 = lax.iota(jnp.int32, 16) * 16
    bias_vec = bb[pl.ds(0, 16)]

    def compute(c, p):
      urows = ubuf.at[p]
      vrows = vbuf.at[p]

      @plsc.parallel_loop(0, ch, step=1, unroll=2)
      def _(i):
        acc = urows[i, pl.ds(0, 16)] * vrows[i, pl.ds(0, 16)] * wvecs[0]
        for j in range(1, 8):
          acc = acc + urows[i, pl.ds(16 * j, 16)] * vrows[i, pl.ds(16 * j, 16)] * wvecs[j]
        accbuf[pl.ds(i * 16, 16)] = acc

      @plsc.parallel_loop(0, ch // 16, step=1)
      def _(g):
        gbase = g * 256
        s = plsc.load_gather(accbuf, [col16 + gbase])
        for l in range(1, 16):
          s = s + plsc.load_gather(accbuf, [col16 + (gbase + l)])
        outv[pl.ds(c * ch + g * 16, 16)] = s + bias_vec

    n_pairs = n_ch // 2

    def pair_body(q, _):
      c0 = 2 * q
      wait(0)
      compute(c0, 0)

      @pl.when(q < n_pairs - 1)
      def _():
        start(c0 + 2, 0)

      wait(1)
      compute(c0 + 1, 1)

      @pl.when(q < n_pairs - 1)
      def _():
        start(c0 + 3, 1)

      return 0

    lax.fori_loop(0, n_pairs, pair_body, 0)
    pltpu.sync_copy(outv, out_hbm.at[pl.ds(base, b_per_w)])

  return k


def kernel(users, items, user_table, item_table, W, b):
  B = users.shape[0]
  info = plsc.get_sparse_core_info()
  nw = info.num_cores * info.num_subcores
  k = _gmf_kernel(B, B // nw, 64, info.num_cores)
  return k(users.astype(jnp.int32), items.astype(jnp.int32),
           user_table, item_table, W.reshape(-1),
           jnp.broadcast_to(b, (16,)))
